# baseline (speedup 1.0000x reference)
import functools

import jax
import jax.numpy as jnp
from jax import lax
from jax.experimental import pallas as pl
from jax.experimental.pallas import tpu as pltpu
from jax.experimental.pallas import tpu_sc as plsc

N = 10000
E = 320000
D = 128
H = 128
G = 64

NC = 2
NS = 16
NW = NC * NS
EPW = E // NW
CH = 125
NCHUNK = EPW // CH
NB = 2
NI = 4
NPAD = 10240
ROWS_PT = NPAD // NS

_mesh = plsc.VectorSubcoreMesh(core_axis_name="c", subcore_axis_name="s")


@functools.partial(
    pl.kernel,
    mesh=_mesh,
    out_type=jax.ShapeDtypeStruct((NC, NPAD, D), jnp.float32),
    scratch_types=[
        pltpu.VMEM((NI, 2, CH), jnp.int32),
        pltpu.VMEM((NB, CH, D), jnp.float32),
        pltpu.VMEM_SHARED((NPAD, D), jnp.float32),
        pltpu.SemaphoreType.DMA((NI,)),
        pltpu.SemaphoreType.DMA((NB,)),
        pltpu.SemaphoreType.DMA((NB,)),
    ],
)
def _sc_agg(h_hbm, ei_hbm, out_hbm, idx_v, rows_v, acc_sh, isem, gsem, ssem):
    cid = lax.axis_index("c")
    sid = lax.axis_index("s")
    wid = cid * NS + sid

    def _ifire(k, sl):
        pltpu.async_copy(ei_hbm.at[wid, k], idx_v.at[sl], isem.at[sl])

    def _iwait(sl):
        pltpu.make_async_copy(ei_hbm.at[wid, 0], idx_v.at[sl],
                              isem.at[sl]).wait()

    def _gfire(sl, b):
        pltpu.async_copy(h_hbm.at[idx_v.at[sl, 0]], rows_v.at[b], gsem.at[b])

    def _gwait(b):
        pltpu.make_async_copy(h_hbm.at[idx_v.at[0, 0]], rows_v.at[b],
                              gsem.at[b]).wait()

    def _sfire(sl, b):
        pltpu.async_copy(rows_v.at[b], acc_sh.at[idx_v.at[sl, 1]], ssem.at[b],
                         add=True)

    def _swait(b):
        pltpu.make_async_copy(rows_v.at[b], acc_sh.at[idx_v.at[0, 1]],
                              ssem.at[b]).wait()

    for sl in range(NI):
        _ifire(sl, sl)

    def _zero_row(i, carry):
        for j in range(D // 16):
            rows_v[0, i, pl.ds(j * 16, 16)] = jnp.zeros((16,), jnp.float32)
        return carry

    lax.fori_loop(0, CH, _zero_row, 0)
    row0 = sid * ROWS_PT
    for z in range(ROWS_PT // CH):
        pltpu.sync_copy(rows_v.at[0], acc_sh.at[pl.ds(row0 + z * CH, CH)])
    rem = ROWS_PT - (ROWS_PT // CH) * CH
    if rem:
        pltpu.sync_copy(
            rows_v.at[0, pl.ds(0, rem)],
            acc_sh.at[pl.ds(row0 + (ROWS_PT // CH) * CH, rem)])
    plsc.subcore_barrier()

    _iwait(0)
    _gfire(0, 0)
    _iwait(1)
    _gfire(1, 1)
    _gwait(0)
    _sfire(0, 0)
    for k in range(1, NI):
        b = k % NB
        _gwait(b)
        _sfire(k, b)
        _swait((k + 1) % NB)
        _ifire(k + 3, (k + 3) % NI)
        _iwait((k + 1) % NI)
        _gfire((k + 1) % NI, (k + 1) % NB)

    def _round(r, carry):
        base = r * NI
        for u in range(NI):
            _gwait(u % NB)
            _sfire(u, u % NB)
            _swait((u + 1) % NB)
            _ifire(base + u + 3, (u + 3) % NI)
            _iwait((u + 1) % NI)
            _gfire((u + 1) % NI, (u + 1) % NB)
        return carry

    lax.fori_loop(1, NCHUNK // NI - 1, _round, 0)

    _gwait(0)
    _sfire(0, 0)
    _swait(1)
    _ifire(NCHUNK - 1, 3)
    _iwait(1)
    _gfire(1, 1)
    for k in range(NCHUNK - 3, NCHUNK - 1):
        b = k % NB
        _gwait(b)
        _sfire(k % NI, b)
        _swait((k + 1) % NB)
        _iwait((k + 1) % NI)
        _gfire((k + 1) % NI, (k + 1) % NB)
    _gwait(1)
    _sfire(3, 1)
    _swait(0)
    _swait(1)

    plsc.subcore_barrier()
    pltpu.sync_copy(acc_sh.at[pl.ds(row0, ROWS_PT)],
                    out_hbm.at[cid, pl.ds(row0, ROWS_PT)])


def _tc_layer_body(h_ref, agg_ref, w1_ref, b1_ref, w2_ref, b2_ref,
                   gam_ref, bet_ref, batch_ref, m_ref, g_ref):
    xsum = h_ref[...] + agg_ref[0, :N] + agg_ref[1, :N]
    a = jnp.dot(xsum, w1_ref[...], preferred_element_type=jnp.float32)
    a = jnp.maximum(a + b1_ref[...], 0.0)
    m = jnp.dot(a, w2_ref[...], preferred_element_type=jnp.float32)
    m = jnp.maximum(m + b2_ref[...], 0.0)
    mu = jnp.mean(m, axis=0)
    var = jnp.mean((m - mu) ** 2, axis=0)
    out = gam_ref[...] * (m - mu) / jnp.sqrt(var + 1e-5) + bet_ref[...]
    m_ref[...] = out
    onehot = (batch_ref[...][None, :]
              == lax.broadcasted_iota(jnp.int32, (G, N), 0)).astype(jnp.float32)
    g_ref[...] = jnp.dot(onehot, out, preferred_element_type=jnp.float32)


_tc_layer = pl.pallas_call(
    _tc_layer_body,
    out_shape=(
        jax.ShapeDtypeStruct((N, H), jnp.float32),
        jax.ShapeDtypeStruct((G, H), jnp.float32),
    ),
)


def kernel(x, edge_index, batch,
           W1_0, b1_0, W2_0, b2_0, gamma_0, beta_0,
           W1_1, b1_1, W2_1, b2_1, gamma_1, beta_1,
           W1_2, b1_2, W2_2, b2_2, gamma_2, beta_2):
    ei = jnp.stack([edge_index[0].reshape(NW, NCHUNK, CH),
                    edge_index[1].reshape(NW, NCHUNK, CH)], axis=2)
    params = [(W1_0, b1_0, W2_0, b2_0, gamma_0, beta_0),
              (W1_1, b1_1, W2_1, b2_1, gamma_1, beta_1),
              (W1_2, b1_2, W2_2, b2_2, gamma_2, beta_2)]
    h = x
    ms, gs = [], []
    for (W1, b1, W2, b2, gamma, beta) in params:
        aggs = _sc_agg(h, ei)
        h, g = _tc_layer(h, aggs, W1, b1, W2, b2, gamma, beta, batch)
        ms.append(h)
        gs.append(g)
    x_patches = jnp.concatenate(ms, axis=1)
    x_global = jnp.concatenate(gs, axis=1)
    return (x_global, x_patches)


# speedup vs baseline: 1.2130x; 1.2130x over previous
import functools

import jax
import jax.numpy as jnp
from jax import lax
from jax.experimental import pallas as pl
from jax.experimental.pallas import tpu as pltpu
from jax.experimental.pallas import tpu_sc as plsc

N = 10000
E = 320000
D = 128
H = 128
G = 64

NC = 2
NS = 16
NW = NC * NS
EPW = E // NW
CH = 50
NCHUNK = EPW // CH
NB = 5
NI = 10
NPAD = 10240
ROWS_PT = NPAD // NS

_mesh = plsc.VectorSubcoreMesh(core_axis_name="c", subcore_axis_name="s")


@functools.partial(
    pl.kernel,
    mesh=_mesh,
    out_type=jax.ShapeDtypeStruct((NC, NPAD, D), jnp.float32),
    scratch_types=[
        pltpu.VMEM((NI, 2, CH), jnp.int32),
        pltpu.VMEM((NB, CH, D), jnp.float32),
        pltpu.VMEM_SHARED((NPAD, D), jnp.float32),
        pltpu.SemaphoreType.DMA((NI,)),
        pltpu.SemaphoreType.DMA((NB,)),
        pltpu.SemaphoreType.DMA((NB,)),
    ],
)
def _sc_agg(h_hbm, ei_hbm, out_hbm, idx_v, rows_v, acc_sh, isem, gsem, ssem):
    cid = lax.axis_index("c")
    sid = lax.axis_index("s")
    wid = cid * NS + sid

    def _ifire(k, sl):
        pltpu.async_copy(ei_hbm.at[wid, k], idx_v.at[sl], isem.at[sl])

    def _iwait(sl):
        pltpu.make_async_copy(ei_hbm.at[wid, 0], idx_v.at[sl],
                              isem.at[sl]).wait()

    def _gfire(sl, b):
        pltpu.async_copy(h_hbm.at[idx_v.at[sl, 0]], rows_v.at[b], gsem.at[b])

    def _gwait(b):
        pltpu.make_async_copy(h_hbm.at[idx_v.at[0, 0]], rows_v.at[b],
                              gsem.at[b]).wait()

    def _sfire(sl, b):
        pltpu.async_copy(rows_v.at[b], acc_sh.at[idx_v.at[sl, 1]], ssem.at[b],
                         add=True)

    def _swait(b):
        pltpu.make_async_copy(rows_v.at[b], acc_sh.at[idx_v.at[0, 1]],
                              ssem.at[b]).wait()

    for sl in range(NI - 1):
        _ifire(sl, sl)

    def _zero_row(i, carry):
        for j in range(D // 16):
            rows_v[0, i, pl.ds(j * 16, 16)] = jnp.zeros((16,), jnp.float32)
        return carry

    lax.fori_loop(0, CH, _zero_row, 0)
    row0 = sid * ROWS_PT
    for z in range(ROWS_PT // CH):
        pltpu.sync_copy(rows_v.at[0], acc_sh.at[pl.ds(row0 + z * CH, CH)])
    rem = ROWS_PT - (ROWS_PT // CH) * CH
    if rem:
        pltpu.sync_copy(
            rows_v.at[0, pl.ds(0, rem)],
            acc_sh.at[pl.ds(row0 + (ROWS_PT // CH) * CH, rem)])
    plsc.subcore_barrier()

    def _step(k, u, first=False, fire_i=True, fire_g=True):
        b = u % NB
        _gwait(b)
        _sfire(u % NI, b)
        if not first:
            _swait((u + 4) % NB)
        if fire_i:
            _ifire(k + 9, (u + 9) % NI)
        if fire_g:
            _iwait((u + 4) % NI)
            _gfire((u + 4) % NI, (u + 4) % NB)

    for j in range(4):
        _iwait(j)
        _gfire(j, j)
    _step(0, 0, first=True)
    for u in range(1, NI):
        _step(u, u)

    def _round(r, carry):
        base = r * NI
        for u in range(NI):
            _step(base + u, u)
        return carry

    lax.fori_loop(1, NCHUNK // NI - 1, _round, 0)

    _step(190, 0)
    for k in range(191, 196):
        _step(k, k % NI, fire_i=False)
    for k in range(196, NCHUNK):
        _step(k, k % NI, fire_i=False, fire_g=False)
    _swait((NCHUNK - 1) % NB)

    plsc.subcore_barrier()
    pltpu.sync_copy(acc_sh.at[pl.ds(row0, ROWS_PT)],
                    out_hbm.at[cid, pl.ds(row0, ROWS_PT)])


def _tc_layer_body(h_ref, agg_ref, w1_ref, b1_ref, w2_ref, b2_ref,
                   gam_ref, bet_ref, batch_ref, m_ref, g_ref):
    xsum = h_ref[...] + agg_ref[0, :N] + agg_ref[1, :N]
    a = jnp.dot(xsum, w1_ref[...], preferred_element_type=jnp.float32)
    a = jnp.maximum(a + b1_ref[...], 0.0)
    m = jnp.dot(a, w2_ref[...], preferred_element_type=jnp.float32)
    m = jnp.maximum(m + b2_ref[...], 0.0)
    mu = jnp.mean(m, axis=0)
    var = jnp.mean((m - mu) ** 2, axis=0)
    out = gam_ref[...] * (m - mu) / jnp.sqrt(var + 1e-5) + bet_ref[...]
    m_ref[...] = out
    onehot = (batch_ref[...][None, :]
              == lax.broadcasted_iota(jnp.int32, (G, N), 0)).astype(jnp.float32)
    g_ref[...] = jnp.dot(onehot, out, preferred_element_type=jnp.float32)


_tc_layer = pl.pallas_call(
    _tc_layer_body,
    out_shape=(
        jax.ShapeDtypeStruct((N, H), jnp.float32),
        jax.ShapeDtypeStruct((G, H), jnp.float32),
    ),
)


def kernel(x, edge_index, batch,
           W1_0, b1_0, W2_0, b2_0, gamma_0, beta_0,
           W1_1, b1_1, W2_1, b2_1, gamma_1, beta_1,
           W1_2, b1_2, W2_2, b2_2, gamma_2, beta_2):
    ei = jnp.stack([edge_index[0].reshape(NW, NCHUNK, CH),
                    edge_index[1].reshape(NW, NCHUNK, CH)], axis=2)
    params = [(W1_0, b1_0, W2_0, b2_0, gamma_0, beta_0),
              (W1_1, b1_1, W2_1, b2_1, gamma_1, beta_1),
              (W1_2, b1_2, W2_2, b2_2, gamma_2, beta_2)]
    h = x
    ms, gs = [], []
    for (W1, b1, W2, b2, gamma, beta) in params:
        aggs = _sc_agg(h, ei)
        h, g = _tc_layer(h, aggs, W1, b1, W2, b2, gamma, beta, batch)
        ms.append(h)
        gs.append(g)
    x_patches = jnp.concatenate(ms, axis=1)
    x_global = jnp.concatenate(gs, axis=1)
    return (x_global, x_patches)


# speedup vs baseline: 1.2130x; 1.0001x over previous
import functools

import jax
import jax.numpy as jnp
from jax import lax
from jax.experimental import pallas as pl
from jax.experimental.pallas import tpu as pltpu
from jax.experimental.pallas import tpu_sc as plsc

N = 10000
E = 320000
D = 128
H = 128
G = 64

NC = 2
NS = 16
NW = NC * NS
EPW = E // NW
CH = 80
NCHUNK = EPW // CH
NB = 3
NI = 6
NPAD = 10240
ROWS_PT = NPAD // NS

_mesh = plsc.VectorSubcoreMesh(core_axis_name="c", subcore_axis_name="s")


@functools.partial(
    pl.kernel,
    mesh=_mesh,
    out_type=jax.ShapeDtypeStruct((NC, NPAD, D), jnp.float32),
    scratch_types=[
        pltpu.VMEM((NI, 2, CH), jnp.int32),
        pltpu.VMEM((NB, CH, D), jnp.float32),
        pltpu.VMEM_SHARED((NPAD, D), jnp.float32),
        pltpu.SemaphoreType.DMA((NI,)),
        pltpu.SemaphoreType.DMA((NB,)),
        pltpu.SemaphoreType.DMA((NB,)),
    ],
)
def _sc_agg(h_hbm, ei_hbm, out_hbm, idx_v, rows_v, acc_sh, isem, gsem, ssem):
    cid = lax.axis_index("c")
    sid = lax.axis_index("s")
    wid = cid * NS + sid

    def _ifire(k, sl):
        pltpu.async_copy(ei_hbm.at[wid, k], idx_v.at[sl], isem.at[sl])

    def _iwait(sl):
        pltpu.make_async_copy(ei_hbm.at[wid, 0], idx_v.at[sl],
                              isem.at[sl]).wait()

    def _gfire(sl, b):
        pltpu.async_copy(h_hbm.at[idx_v.at[sl, 0]], rows_v.at[b], gsem.at[b])

    def _gwait(b):
        pltpu.make_async_copy(h_hbm.at[idx_v.at[0, 0]], rows_v.at[b],
                              gsem.at[b]).wait()

    def _sfire(sl, b):
        pltpu.async_copy(rows_v.at[b], acc_sh.at[idx_v.at[sl, 1]], ssem.at[b],
                         add=True)

    def _swait(b):
        pltpu.make_async_copy(rows_v.at[b], acc_sh.at[idx_v.at[0, 1]],
                              ssem.at[b]).wait()

    def _zero_row(i, carry):
        for j in range(D // 16):
            rows_v[0, i, pl.ds(j * 16, 16)] = jnp.zeros((16,), jnp.float32)
        return carry

    lax.fori_loop(0, CH, _zero_row, 0)
    row0 = sid * ROWS_PT
    for z in range(ROWS_PT // CH):
        pltpu.sync_copy(rows_v.at[0], acc_sh.at[pl.ds(row0 + z * CH, CH)])
    plsc.subcore_barrier()

    def _step(k, u, first=False, fire_i=True, fire_g=True):
        b = u % NB
        _gwait(b)
        _sfire(u % NI, b)
        if not first:
            _swait((u + 2) % NB)
        if fire_i:
            _ifire(k + 5, (u + 5) % NI)
        if fire_g:
            _iwait((u + 2) % NI)
            _gfire((u + 2) % NI, (u + 2) % NB)

    for sl in range(5):
        _ifire(sl, sl)
    _iwait(0)
    _gfire(0, 0)
    _iwait(1)
    _gfire(1, 1)
    _step(0, 0, first=True)
    for u in range(1, 6):
        _step(u, u)

    def _round(i2, carry):
        base = i2 * 6
        for u in range(6):
            _step(base + u, u)
        return carry

    lax.fori_loop(1, 20, _round, 0)

    _step(120, 120 % NI, fire_i=False)
    _step(121, 121 % NI, fire_i=False)
    _step(122, 122 % NI, fire_i=False)
    _step(123, 123 % NI, fire_i=False, fire_g=False)
    _step(124, 124 % NI, fire_i=False, fire_g=False)
    _swait(124 % NB)

    plsc.subcore_barrier()
    pltpu.sync_copy(acc_sh.at[pl.ds(row0, ROWS_PT)],
                    out_hbm.at[cid, pl.ds(row0, ROWS_PT)])


def _tc_layer_body(h_ref, agg_ref, w1_ref, b1_ref, w2_ref, b2_ref,
                   gam_ref, bet_ref, batch_ref, m_ref, g_ref):
    xsum = h_ref[...] + agg_ref[0, :N] + agg_ref[1, :N]
    a = jnp.dot(xsum, w1_ref[...], preferred_element_type=jnp.float32)
    a = jnp.maximum(a + b1_ref[...], 0.0)
    m = jnp.dot(a, w2_ref[...], preferred_element_type=jnp.float32)
    m = jnp.maximum(m + b2_ref[...], 0.0)
    mu = jnp.mean(m, axis=0)
    var = jnp.mean((m - mu) ** 2, axis=0)
    out = gam_ref[...] * (m - mu) / jnp.sqrt(var + 1e-5) + bet_ref[...]
    m_ref[...] = out
    onehot = (batch_ref[...][None, :]
              == lax.broadcasted_iota(jnp.int32, (G, N), 0)).astype(jnp.float32)
    g_ref[...] = jnp.dot(onehot, out, preferred_element_type=jnp.float32)


_tc_layer = pl.pallas_call(
    _tc_layer_body,
    out_shape=(
        jax.ShapeDtypeStruct((N, H), jnp.float32),
        jax.ShapeDtypeStruct((G, H), jnp.float32),
    ),
)


def kernel(x, edge_index, batch,
           W1_0, b1_0, W2_0, b2_0, gamma_0, beta_0,
           W1_1, b1_1, W2_1, b2_1, gamma_1, beta_1,
           W1_2, b1_2, W2_2, b2_2, gamma_2, beta_2):
    ei = jnp.stack([edge_index[0].reshape(NW, NCHUNK, CH),
                    edge_index[1].reshape(NW, NCHUNK, CH)], axis=2)
    params = [(W1_0, b1_0, W2_0, b2_0, gamma_0, beta_0),
              (W1_1, b1_1, W2_1, b2_1, gamma_1, beta_1),
              (W1_2, b1_2, W2_2, b2_2, gamma_2, beta_2)]
    h = x
    ms, gs = [], []
    for (W1, b1, W2, b2, gamma, beta) in params:
        aggs = _sc_agg(h, ei)
        h, g = _tc_layer(h, aggs, W1, b1, W2, b2, gamma, beta, batch)
        ms.append(h)
        gs.append(g)
    x_patches = jnp.concatenate(ms, axis=1)
    x_global = jnp.concatenate(gs, axis=1)
    return (x_global, x_patches)
